# baseline (speedup 1.0000x reference)
import functools

import jax
import jax.numpy as jnp
from jax import lax
from jax.experimental import pallas as pl
from jax.experimental.pallas import tpu as pltpu
from jax.experimental.pallas import tpu_sc as plsc

N_NODES = 10000
N_EDGES = 320000
HIDDEN = 128

_NUM_WORKERS = 32
_EDGES_PER_WORKER = N_EDGES // _NUM_WORKERS
_LANES = 16


_BLK = 128
_N_PAD = 10240
_N_BLKS = (N_NODES + _BLK - 1) // _BLK


def _scores_body(x_ref, w_ref, s_ref):
    g = pl.program_id(0)
    s01 = jnp.dot(w_ref[...], x_ref[...].T,
                  preferred_element_type=jnp.float32)
    s_ref[pl.ds(g * _BLK, _BLK)] = s01[0, :]
    s_ref[pl.ds(_N_PAD + g * _BLK, _BLK)] = s01[1, :]


def _node_scores(x, w):
    return pl.pallas_call(
        _scores_body,
        grid=(_N_BLKS,),
        in_specs=[
            pl.BlockSpec((_BLK, HIDDEN), lambda g: (g, 0)),
            pl.BlockSpec((2, HIDDEN), lambda g: (0, 0)),
        ],
        out_specs=pl.BlockSpec((2 * _N_PAD,), lambda g: (0,)),
        out_shape=jax.ShapeDtypeStruct((2 * _N_PAD,), jnp.float32),
    )(x, w)


def _make_edge_kernel():
    mesh = plsc.VectorSubcoreMesh(core_axis_name="c", subcore_axis_name="s")

    @functools.partial(
        pl.kernel,
        mesh=mesh,
        out_type=jax.ShapeDtypeStruct((N_EDGES,), jnp.float32),
        compiler_params=pltpu.CompilerParams(needs_layout_passes=False),
        scratch_types=[
            pltpu.VMEM((_EDGES_PER_WORKER,), jnp.int32),
            pltpu.VMEM((_EDGES_PER_WORKER,), jnp.int32),
            pltpu.VMEM((2 * _N_PAD,), jnp.float32),
            pltpu.VMEM((_EDGES_PER_WORKER,), jnp.float32),
            pltpu.SemaphoreType.DMA,
            pltpu.SemaphoreType.DMA,
            pltpu.SemaphoreType.DMA,
        ],
    )
    def edge_kernel(ei_hbm, s_hbm, out_hbm, row_v, col_v, s_v, out_v,
                    sem0, sem1, sem2):
        wid = lax.axis_index("s") * 2 + lax.axis_index("c")
        base = wid * _EDGES_PER_WORKER
        cp0 = pltpu.async_copy(ei_hbm.at[pl.ds(base, _EDGES_PER_WORKER)], row_v, sem0)
        cp1 = pltpu.async_copy(ei_hbm.at[pl.ds(N_EDGES + base, _EDGES_PER_WORKER)], col_v, sem1)
        cp2 = pltpu.async_copy(s_hbm, s_v, sem2)
        cp0.wait()
        cp1.wait()
        cp2.wait()

        @plsc.parallel_loop(0, _EDGES_PER_WORKER, step=_LANES, unroll=8)
        def body(off):
            r = row_v[pl.ds(off, _LANES)]
            c = col_v[pl.ds(off, _LANES)]
            a = plsc.load_gather(s_v, [r])
            b = plsc.load_gather(s_v, [c + _N_PAD])
            z = a + b
            out_v[pl.ds(off, _LANES)] = 1.0 / (1.0 + jnp.exp(-z))

        pltpu.sync_copy(out_v, out_hbm.at[pl.ds(base, _EDGES_PER_WORKER)])

    return edge_kernel


_edge_kernel = _make_edge_kernel()


def kernel(edge_index, x, att_weight):
    ei_flat = edge_index.astype(jnp.int32).reshape(2 * N_EDGES)
    w = att_weight.reshape(2, HIDDEN)
    s = _node_scores(x, w)
    return _edge_kernel(ei_flat, s).reshape(N_EDGES, 1)


# speedup vs baseline: 1.8376x; 1.8376x over previous
import functools

import jax
import jax.numpy as jnp
from jax import lax
from jax.experimental import pallas as pl
from jax.experimental.pallas import tpu as pltpu
from jax.experimental.pallas import tpu_sc as plsc

N_NODES = 10000
N_EDGES = 320000
HIDDEN = 128

_NUM_WORKERS = 32
_EDGES_PER_WORKER = N_EDGES // _NUM_WORKERS
_LANES = 16


_N_PAD = 10240


def _scores_body(x_ref, w_ref, s_ref):
    x = x_ref[...]
    w = w_ref[...]
    s0 = jnp.sum(x * w[0:1, :], axis=1)
    s1 = jnp.sum(x * w[1:2, :], axis=1)
    s_ref[pl.ds(0, N_NODES)] = s0
    s_ref[pl.ds(_N_PAD, N_NODES)] = s1


def _node_scores(x, w):
    return pl.pallas_call(
        _scores_body,
        out_shape=jax.ShapeDtypeStruct((2 * _N_PAD,), jnp.float32),
    )(x, w)


_TILE = 128
_N_TILES = N_EDGES // _TILE
_T_MAIN = _N_TILES // _NUM_WORKERS
_E_MAIN = _T_MAIN * _TILE
_T_EXTRA = _N_TILES - _T_MAIN * _NUM_WORKERS


def _make_edge_kernel():
    mesh = plsc.VectorSubcoreMesh(core_axis_name="c", subcore_axis_name="s")

    @functools.partial(
        pl.kernel,
        mesh=mesh,
        out_type=jax.ShapeDtypeStruct((N_EDGES,), jnp.float32),
        compiler_params=pltpu.CompilerParams(needs_layout_passes=False),
        scratch_types=[
            pltpu.VMEM((2, _E_MAIN), jnp.int32),
            pltpu.VMEM((2, _TILE), jnp.int32),
            pltpu.VMEM((2 * _N_PAD,), jnp.float32),
            pltpu.VMEM((_E_MAIN,), jnp.float32),
            pltpu.VMEM((_TILE,), jnp.float32),
            pltpu.SemaphoreType.DMA,
            pltpu.SemaphoreType.DMA,
            pltpu.SemaphoreType.DMA,
        ],
    )
    def edge_kernel(ei_hbm, s_hbm, out_hbm, ei_v, ei_fix_v, s_v, out_v,
                    out_fix_v, sem0, sem1, sem2):
        wid = lax.axis_index("s") * 2 + lax.axis_index("c")
        base = wid * _E_MAIN
        cp0 = pltpu.async_copy(ei_hbm.at[:, pl.ds(base, _E_MAIN)], ei_v, sem0)
        cp2 = pltpu.async_copy(s_hbm, s_v, sem2)

        fix_base = (_N_TILES - _T_EXTRA + wid) * _TILE

        @pl.when(wid < _T_EXTRA)
        def _():
            pltpu.async_copy(ei_hbm.at[:, pl.ds(fix_base, _TILE)], ei_fix_v,
                             sem1).wait()

        cp0.wait()
        cp2.wait()

        @plsc.parallel_loop(0, _E_MAIN, step=_LANES, unroll=8)
        def body(off):
            r = ei_v[0, pl.ds(off, _LANES)]
            c = ei_v[1, pl.ds(off, _LANES)]
            a = plsc.load_gather(s_v, [r])
            b = plsc.load_gather(s_v, [c + _N_PAD])
            z = a + b
            out_v[pl.ds(off, _LANES)] = 1.0 / (1.0 + jnp.exp(-z))

        @pl.when(wid < _T_EXTRA)
        def _():
            @plsc.parallel_loop(0, _TILE, step=_LANES, unroll=8)
            def fix_body(off):
                r = ei_fix_v[0, pl.ds(off, _LANES)]
                c = ei_fix_v[1, pl.ds(off, _LANES)]
                a = plsc.load_gather(s_v, [r])
                b = plsc.load_gather(s_v, [c + _N_PAD])
                z = a + b
                out_fix_v[pl.ds(off, _LANES)] = 1.0 / (1.0 + jnp.exp(-z))

            pltpu.sync_copy(out_fix_v, out_hbm.at[pl.ds(fix_base, _TILE)])

        pltpu.sync_copy(out_v, out_hbm.at[pl.ds(base, _E_MAIN)])

    return edge_kernel


_edge_kernel = _make_edge_kernel()


def kernel(edge_index, x, att_weight):
    ei = edge_index.astype(jnp.int32)
    w = att_weight.reshape(2, HIDDEN)
    s = _node_scores(x, w)
    return _edge_kernel(ei, s).reshape(N_EDGES, 1)


# speedup vs baseline: 2.2282x; 1.2126x over previous
import functools

import jax
import jax.numpy as jnp
from jax import lax
from jax.experimental import pallas as pl
from jax.experimental.pallas import tpu as pltpu
from jax.experimental.pallas import tpu_sc as plsc

N_NODES = 10000
N_EDGES = 320000
HIDDEN = 128

_NUM_WORKERS = 32
_EDGES_PER_WORKER = N_EDGES // _NUM_WORKERS
_LANES = 16


_N_PAD = 10240


def _scores_body(x_ref, w_ref, s_ref):
    s01 = jnp.dot(w_ref[...], x_ref[...].T,
                  preferred_element_type=jnp.float32)
    s_ref[pl.ds(0, N_NODES)] = s01[0, :]
    s_ref[pl.ds(_N_PAD, N_NODES)] = s01[1, :]


def _node_scores(x, w):
    return pl.pallas_call(
        _scores_body,
        out_shape=jax.ShapeDtypeStruct((2 * _N_PAD,), jnp.float32),
    )(x, w)


_TILE = 128
_N_TILES = N_EDGES // _TILE
_T_MAIN = _N_TILES // _NUM_WORKERS
_E_MAIN = _T_MAIN * _TILE
_T_EXTRA = _N_TILES - _T_MAIN * _NUM_WORKERS


def _make_edge_kernel():
    mesh = plsc.VectorSubcoreMesh(core_axis_name="c", subcore_axis_name="s")

    @functools.partial(
        pl.kernel,
        mesh=mesh,
        out_type=jax.ShapeDtypeStruct((N_EDGES,), jnp.float32),
        compiler_params=pltpu.CompilerParams(needs_layout_passes=False),
        scratch_types=[
            pltpu.VMEM((2, _E_MAIN), jnp.int32),
            pltpu.VMEM((2, _TILE), jnp.int32),
            pltpu.VMEM((2 * _N_PAD,), jnp.float32),
            pltpu.VMEM((_E_MAIN,), jnp.float32),
            pltpu.VMEM((_TILE,), jnp.float32),
            pltpu.SemaphoreType.DMA,
            pltpu.SemaphoreType.DMA,
            pltpu.SemaphoreType.DMA,
        ],
    )
    def edge_kernel(ei_hbm, s_hbm, out_hbm, ei_v, ei_fix_v, s_v, out_v,
                    out_fix_v, sem0, sem1, sem2):
        wid = lax.axis_index("s") * 2 + lax.axis_index("c")
        base = wid * _E_MAIN
        cp0 = pltpu.async_copy(ei_hbm.at[:, pl.ds(base, _E_MAIN)], ei_v, sem0)
        cp2 = pltpu.async_copy(s_hbm, s_v, sem2)

        fix_base = (_N_TILES - _T_EXTRA + wid) * _TILE

        @pl.when(wid < _T_EXTRA)
        def _():
            pltpu.async_copy(ei_hbm.at[:, pl.ds(fix_base, _TILE)], ei_fix_v,
                             sem1).wait()

        cp0.wait()
        cp2.wait()

        @plsc.parallel_loop(0, _E_MAIN, step=_LANES, unroll=8)
        def body(off):
            r = ei_v[0, pl.ds(off, _LANES)]
            c = ei_v[1, pl.ds(off, _LANES)]
            a = plsc.load_gather(s_v, [r])
            b = plsc.load_gather(s_v, [c + _N_PAD])
            z = a + b
            out_v[pl.ds(off, _LANES)] = 1.0 / (1.0 + jnp.exp(-z))

        @pl.when(wid < _T_EXTRA)
        def _():
            @plsc.parallel_loop(0, _TILE, step=_LANES, unroll=8)
            def fix_body(off):
                r = ei_fix_v[0, pl.ds(off, _LANES)]
                c = ei_fix_v[1, pl.ds(off, _LANES)]
                a = plsc.load_gather(s_v, [r])
                b = plsc.load_gather(s_v, [c + _N_PAD])
                z = a + b
                out_fix_v[pl.ds(off, _LANES)] = 1.0 / (1.0 + jnp.exp(-z))

            pltpu.sync_copy(out_fix_v, out_hbm.at[pl.ds(fix_base, _TILE)])

        pltpu.sync_copy(out_v, out_hbm.at[pl.ds(base, _E_MAIN)])

    return edge_kernel


_edge_kernel = _make_edge_kernel()


def kernel(edge_index, x, att_weight):
    ei = edge_index.astype(jnp.int32)
    w = att_weight.reshape(2, HIDDEN)
    s = _node_scores(x, w)
    return _edge_kernel(ei, s).reshape(N_EDGES, 1)
